# Initial kernel scaffold; baseline (speedup 1.0000x reference)
import functools
import math

import jax
import jax.numpy as jnp
from jax import lax
from jax.experimental import pallas as pl
from jax.experimental.pallas import tpu as pltpu
from jax.experimental.pallas import tpu_sc as plsc

NN = 10000
EE = 160000
DD = 256
HH = 4
CC = DD // HH

NC = 2
NS = 16
LANES = 16


def _q_body(x_ref, wq_ref, q_ref):
    q_ref[...] = jnp.dot(x_ref[...], wq_ref[...],
                         preferred_element_type=jnp.float32)


def _tc_query(x, Wq):
    BN = 1250
    return pl.pallas_call(
        _q_body,
        grid=(NN // BN,),
        in_specs=[pl.BlockSpec((BN, DD), lambda i: (i, 0)),
                  pl.BlockSpec((DD, DD), lambda i: (0, 0))],
        out_specs=pl.BlockSpec((BN, DD), lambda i: (i, 0)),
        out_shape=jax.ShapeDtypeStruct((NN, DD), jnp.float32),
    )(x, Wq)


def _sc_gather(x, query, dst, src):
    mesh = plsc.VectorSubcoreMesh(core_axis_name="c", subcore_axis_name="s")
    EW = EE // (NC * NS)
    CH = 128
    NF = EW // CH
    TL = EW - NF * CH

    @functools.partial(
        pl.kernel,
        out_type=(jax.ShapeDtypeStruct((EE, DD), jnp.float32),
                  jax.ShapeDtypeStruct((EE, DD), jnp.float32)),
        mesh=mesh,
        scratch_types=[pltpu.VMEM((CH,), jnp.int32),
                       pltpu.VMEM((CH, DD), jnp.float32),
                       pltpu.VMEM((TL,), jnp.int32),
                       pltpu.VMEM((TL, DD), jnp.float32),
                       pltpu.SemaphoreType.DMA],
    )
    def k(x_hbm, q_hbm, dst_hbm, src_hbm, xd_hbm, qs_hbm,
          idx_v, rows_v, idxt_v, rowst_v, sem):
        wid = lax.axis_index("s") * NC + lax.axis_index("c")
        base = wid * EW

        def chunk(off, idxr, rowsr, tab_hbm, ind_hbm, out_hbm, sz):
            pltpu.sync_copy(ind_hbm.at[pl.ds(off, sz)], idxr)
            pltpu.async_copy(tab_hbm.at[idxr], rowsr, sem).wait()
            pltpu.sync_copy(rowsr, out_hbm.at[pl.ds(off, sz)])

        def body(i, carry):
            off = base + i * CH
            chunk(off, idx_v, rows_v, x_hbm, dst_hbm, xd_hbm, CH)
            chunk(off, idx_v, rows_v, q_hbm, src_hbm, qs_hbm, CH)
            return carry

        lax.fori_loop(0, NF, body, 0)
        offt = base + NF * CH
        chunk(offt, idxt_v, rowst_v, x_hbm, dst_hbm, xd_hbm, TL)
        chunk(offt, idxt_v, rowst_v, q_hbm, src_hbm, qs_hbm, TL)

    return k(x, query, dst, src)


def _att_body(ea_ref, xd_ref, qs_ref, wk_ref, wv_ref, we_ref, be_ref,
              sh_ref, v_ref, eo_ref, att_ref):
    z = ea_ref[...] * xd_ref[...]
    kk = jnp.dot(z, wk_ref[...], preferred_element_type=jnp.float32)
    v_ref[...] = jnp.dot(z, wv_ref[...], preferred_element_type=jnp.float32)
    eo_ref[...] = (jnp.dot(z, we_ref[...], preferred_element_type=jnp.float32)
                   + be_ref[...])
    att_ref[...] = jnp.dot(qs_ref[...] * kk, sh_ref[...],
                           preferred_element_type=jnp.float32)


def _tc_proj(edge_attr, xd, qs, Wk, Wv, We, be):
    BE = 800
    shead = jnp.repeat(jnp.eye(HH, dtype=jnp.float32), CC, axis=0) \
        * (1.0 / math.sqrt(CC))
    be2 = be.reshape(1, DD)
    return pl.pallas_call(
        _att_body,
        grid=(EE // BE,),
        in_specs=[pl.BlockSpec((BE, DD), lambda i: (i, 0)),
                  pl.BlockSpec((BE, DD), lambda i: (i, 0)),
                  pl.BlockSpec((BE, DD), lambda i: (i, 0)),
                  pl.BlockSpec((DD, DD), lambda i: (0, 0)),
                  pl.BlockSpec((DD, DD), lambda i: (0, 0)),
                  pl.BlockSpec((DD, DD), lambda i: (0, 0)),
                  pl.BlockSpec((1, DD), lambda i: (0, 0)),
                  pl.BlockSpec((DD, HH), lambda i: (0, 0))],
        out_specs=[pl.BlockSpec((BE, DD), lambda i: (i, 0)),
                   pl.BlockSpec((BE, DD), lambda i: (i, 0)),
                   pl.BlockSpec((BE, HH), lambda i: (i, 0))],
        out_shape=[jax.ShapeDtypeStruct((EE, DD), jnp.float32),
                   jax.ShapeDtypeStruct((EE, DD), jnp.float32),
                   jax.ShapeDtypeStruct((EE, HH), jnp.float32)],
    )(edge_attr, xd, qs, Wk, Wv, We, be2, shead)


def _sc_softmax(att, src):
    mesh = plsc.VectorSubcoreMesh(core_axis_name="c", subcore_axis_name="s",
                                  num_cores=1)
    ET = EE // NS
    CH = 128
    NF = ET // CH
    TL = ET - NF * CH
    RT = 624
    NG = (ET * HH) // LANES

    @functools.partial(
        pl.kernel,
        out_type=jax.ShapeDtypeStruct((EE, HH), jnp.float32),
        mesh=mesh,
        scratch_types=[
            pltpu.VMEM((ET, HH), jnp.float32),
            pltpu.VMEM((NN, HH), jnp.float32),
            pltpu.VMEM((ET,), jnp.int32),
            pltpu.VMEM((CH,), jnp.int32),
            pltpu.VMEM((TL,), jnp.int32),
            pltpu.VMEM_SHARED((NN, HH), jnp.float32),
            pltpu.VMEM_SHARED((NN, HH), jnp.float32),
        ],
    )
    def k(att_hbm, src_hbm, coeff_hbm, av, tab, srcv, idx, idxt,
          mtab_s, den_s):
        t = lax.axis_index("s")
        base = t * ET
        li = lax.iota(jnp.int32, LANES)
        zero = jnp.zeros((LANES,), jnp.float32)

        def zbody(g, carry):
            p = g * LANES + li
            plsc.store_scatter(av, [p // HH, p % HH], zero)
            return carry
        lax.fori_loop(0, (640 * HH) // LANES, zbody, 0)
        pltpu.sync_copy(av.at[pl.ds(0, RT), :],
                        den_s.at[pl.ds(t * RT, RT), :])

        @pl.when(t == NS - 1)
        def _():
            pltpu.sync_copy(av.at[pl.ds(0, 16), :],
                            den_s.at[pl.ds(NS * RT, 16), :])

        pltpu.sync_copy(att_hbm.at[pl.ds(base, ET), :], av)
        pltpu.sync_copy(src_hbm.at[pl.ds(base, ET)], srcv)

        def mscat(i, carry):
            off = i * CH
            pltpu.sync_copy(src_hbm.at[pl.ds(base + off, CH)], idx)
            pltpu.sync_copy(av.at[pl.ds(off, CH), :], mtab_s.at[idx])
            return carry
        lax.fori_loop(0, NF, mscat, 0)
        pltpu.sync_copy(src_hbm.at[pl.ds(base + NF * CH, TL)], idxt)
        pltpu.sync_copy(av.at[pl.ds(NF * CH, TL), :], mtab_s.at[idxt])

        plsc.subcore_barrier()
        pltpu.sync_copy(mtab_s, tab)

        def exbody(g, carry):
            p = g * LANES + li
            el = p // HH
            hh = p % HH
            s = plsc.load_gather(srcv, [el])
            m = plsc.load_gather(tab, [s, hh])
            a = plsc.load_gather(av, [el, hh])
            plsc.store_scatter(av, [el, hh], jnp.exp(a - m))
            return carry
        lax.fori_loop(0, NG, exbody, 0)

        def dscat(i, carry):
            off = i * CH
            pltpu.sync_copy(src_hbm.at[pl.ds(base + off, CH)], idx)
            pltpu.sync_copy(av.at[pl.ds(off, CH), :], den_s.at[idx],
                            add=True)
            return carry
        lax.fori_loop(0, NF, dscat, 0)
        pltpu.sync_copy(src_hbm.at[pl.ds(base + NF * CH, TL)], idxt)
        pltpu.sync_copy(av.at[pl.ds(NF * CH, TL), :], den_s.at[idxt],
                        add=True)

        plsc.subcore_barrier()
        pltpu.sync_copy(den_s, tab)

        def cbody(g, carry):
            p = g * LANES + li
            el = p // HH
            hh = p % HH
            s = plsc.load_gather(srcv, [el])
            d = plsc.load_gather(tab, [s, hh])
            e = plsc.load_gather(av, [el, hh])
            plsc.store_scatter(av, [el, hh], e / (d + 1e-16))
            return carry
        lax.fori_loop(0, NG, cbody, 0)
        pltpu.sync_copy(av, coeff_hbm.at[pl.ds(base, ET), :])

    return k(att, src)


def _w_body(coeff_ref, v_ref, ex_ref, w_ref):
    w = v_ref[...] * jnp.dot(coeff_ref[...], ex_ref[...],
                             preferred_element_type=jnp.float32)
    w_ref[:, 0, :] = w[:, :DD // 2]
    w_ref[:, 1, :] = w[:, DD // 2:]


def _tc_scale(coeff, v):
    BE = 800
    expand = jnp.repeat(jnp.eye(HH, dtype=jnp.float32), CC, axis=1)
    return pl.pallas_call(
        _w_body,
        grid=(EE // BE,),
        in_specs=[pl.BlockSpec((BE, HH), lambda i: (i, 0)),
                  pl.BlockSpec((BE, DD), lambda i: (i, 0)),
                  pl.BlockSpec((HH, DD), lambda i: (0, 0))],
        out_specs=pl.BlockSpec((BE, 2, DD // 2), lambda i: (i, 0, 0)),
        out_shape=jax.ShapeDtypeStruct((EE, 2, DD // 2), jnp.float32),
    )(coeff, v, expand)


def _sc_scatter_out(w2, src):
    mesh = plsc.VectorSubcoreMesh(core_axis_name="c", subcore_axis_name="s")
    ET = EE // NS
    CH = 128
    NF = ET // CH
    TL = ET - NF * CH
    RT = 624
    HW = DD // 2

    @functools.partial(
        pl.kernel,
        out_type=jax.ShapeDtypeStruct((NN, 2, HW), jnp.float32),
        mesh=mesh,
        scratch_types=[
            pltpu.VMEM((CH, HW), jnp.float32),
            pltpu.VMEM((TL, HW), jnp.float32),
            pltpu.VMEM((CH,), jnp.int32),
            pltpu.VMEM((TL,), jnp.int32),
            pltpu.VMEM_SHARED((NN, HW), jnp.float32),
        ],
    )
    def k(w_hbm, src_hbm, out_hbm, wbuf, wbuft, idx, idxt, acc_s):
        c = lax.axis_index("c")
        t = lax.axis_index("s")
        zero = jnp.zeros((LANES,), jnp.float32)

        def zrow(r, carry):
            def zcol(j, carry2):
                wbuf[r, pl.ds(j * LANES, LANES)] = zero
                return carry2
            return lax.fori_loop(0, HW // LANES, zcol, carry)
        lax.fori_loop(0, CH, zrow, 0)

        def zcopy(i, carry):
            pltpu.sync_copy(wbuf, acc_s.at[pl.ds(t * RT + i * CH, CH), :])
            return carry
        lax.fori_loop(0, 4, zcopy, 0)
        pltpu.sync_copy(wbuf.at[pl.ds(0, RT - 4 * CH), :],
                        acc_s.at[pl.ds(t * RT + 4 * CH, RT - 4 * CH), :])

        @pl.when(t == NS - 1)
        def _():
            pltpu.sync_copy(wbuf.at[pl.ds(0, 16), :],
                            acc_s.at[pl.ds(NS * RT, 16), :])

        plsc.subcore_barrier()

        base = t * ET

        def scat(i, carry):
            off = base + i * CH
            pltpu.sync_copy(src_hbm.at[pl.ds(off, CH)], idx)
            pltpu.sync_copy(w_hbm.at[pl.ds(off, CH), c, :], wbuf)
            pltpu.sync_copy(wbuf, acc_s.at[idx], add=True)
            return carry
        lax.fori_loop(0, NF, scat, 0)
        offt = base + NF * CH
        pltpu.sync_copy(src_hbm.at[pl.ds(offt, TL)], idxt)
        pltpu.sync_copy(w_hbm.at[pl.ds(offt, TL), c, :], wbuft)
        pltpu.sync_copy(wbuft, acc_s.at[idxt], add=True)

        plsc.subcore_barrier()

        pltpu.sync_copy(acc_s.at[pl.ds(t * RT, RT), :],
                        out_hbm.at[pl.ds(t * RT, RT), c, :])

        @pl.when(t == NS - 1)
        def _():
            pltpu.sync_copy(acc_s.at[pl.ds(NS * RT, 16), :],
                            out_hbm.at[pl.ds(NS * RT, 16), c, :])

    return k(w2, src)


def kernel(x, edge_index, edge_attr, Wq, Wk, Wv, We, be):
    src = edge_index[0]
    dst = edge_index[1]
    query = _tc_query(x, Wq)
    xd, qs = _sc_gather(x, query, dst, src)
    v, eout, att = _tc_proj(edge_attr, xd, qs, Wk, Wv, We, be)
    coeff = _sc_softmax(att, src)
    w2 = _tc_scale(coeff, v)
    out2 = _sc_scatter_out(w2, src)
    return out2.reshape(NN, DD), eout


# speedup vs baseline: 10.0674x; 10.0674x over previous
import functools
import math

import jax
import jax.numpy as jnp
from jax import lax
from jax.experimental import pallas as pl
from jax.experimental.pallas import tpu as pltpu
from jax.experimental.pallas import tpu_sc as plsc

NN = 10000
EE = 160000
DD = 256
HH = 4
CC = DD // HH

NC = 2
NS = 16
LANES = 16


def _q_body(x_ref, wq_ref, q_ref):
    q_ref[...] = jnp.dot(x_ref[...], wq_ref[...],
                         preferred_element_type=jnp.float32)


def _tc_query(x, Wq):
    BN = 1000
    return pl.pallas_call(
        _q_body,
        grid=(NN // BN,),
        in_specs=[pl.BlockSpec((BN, DD), lambda i: (i, 0)),
                  pl.BlockSpec((DD, DD), lambda i: (0, 0))],
        out_specs=pl.BlockSpec((BN, DD), lambda i: (i, 0)),
        out_shape=jax.ShapeDtypeStruct((NN, DD), jnp.float32),
    )(x, Wq)


def _sc_gather(x, query, dst, src):
    mesh = plsc.VectorSubcoreMesh(core_axis_name="c", subcore_axis_name="s")
    EW = EE // (NC * NS)
    CH = 128
    NF = EW // CH
    TL = EW - NF * CH

    @functools.partial(
        pl.kernel,
        out_type=(jax.ShapeDtypeStruct((EE, DD), jnp.float32),
                  jax.ShapeDtypeStruct((EE, DD), jnp.float32)),
        mesh=mesh,
        scratch_types=[pltpu.VMEM((CH,), jnp.int32),
                       pltpu.VMEM((CH, DD), jnp.float32),
                       pltpu.VMEM((TL,), jnp.int32),
                       pltpu.VMEM((TL, DD), jnp.float32),
                       pltpu.SemaphoreType.DMA],
    )
    def k(x_hbm, q_hbm, dst_hbm, src_hbm, xd_hbm, qs_hbm,
          idx_v, rows_v, idxt_v, rowst_v, sem):
        wid = lax.axis_index("s") * NC + lax.axis_index("c")
        base = wid * EW

        def chunk(off, idxr, rowsr, tab_hbm, ind_hbm, out_hbm, sz):
            pltpu.sync_copy(ind_hbm.at[pl.ds(off, sz)], idxr)
            pltpu.async_copy(tab_hbm.at[idxr], rowsr, sem).wait()
            pltpu.sync_copy(rowsr, out_hbm.at[pl.ds(off, sz)])

        def body(i, carry):
            off = base + i * CH
            chunk(off, idx_v, rows_v, x_hbm, dst_hbm, xd_hbm, CH)
            chunk(off, idx_v, rows_v, q_hbm, src_hbm, qs_hbm, CH)
            return carry

        lax.fori_loop(0, NF, body, 0)
        offt = base + NF * CH
        chunk(offt, idxt_v, rowst_v, x_hbm, dst_hbm, xd_hbm, TL)
        chunk(offt, idxt_v, rowst_v, q_hbm, src_hbm, qs_hbm, TL)

    return k(x, query, dst, src)


def _att_body(ea_ref, xd_ref, qs_ref, wk_ref, wv_ref, we_ref, be_ref,
              sh_ref, v_ref, eo_ref, att_ref):
    z = ea_ref[...] * xd_ref[...]
    kk = jnp.dot(z, wk_ref[...], preferred_element_type=jnp.float32)
    v_ref[...] = jnp.dot(z, wv_ref[...], preferred_element_type=jnp.float32)
    eo_ref[...] = (jnp.dot(z, we_ref[...], preferred_element_type=jnp.float32)
                   + be_ref[...])
    att = jnp.dot(qs_ref[...] * kk, sh_ref[...],
                  preferred_element_type=jnp.float32)
    att_ref[...] = att.T


def _tc_proj(edge_attr, xd, qs, Wk, Wv, We, be):
    BE = 640
    shead = jnp.repeat(jnp.eye(HH, dtype=jnp.float32), CC, axis=0) \
        * (1.0 / math.sqrt(CC))
    be2 = be.reshape(1, DD)
    return pl.pallas_call(
        _att_body,
        grid=(EE // BE,),
        in_specs=[pl.BlockSpec((BE, DD), lambda i: (i, 0)),
                  pl.BlockSpec((BE, DD), lambda i: (i, 0)),
                  pl.BlockSpec((BE, DD), lambda i: (i, 0)),
                  pl.BlockSpec((DD, DD), lambda i: (0, 0)),
                  pl.BlockSpec((DD, DD), lambda i: (0, 0)),
                  pl.BlockSpec((DD, DD), lambda i: (0, 0)),
                  pl.BlockSpec((1, DD), lambda i: (0, 0)),
                  pl.BlockSpec((DD, HH), lambda i: (0, 0))],
        out_specs=[pl.BlockSpec((BE, DD), lambda i: (i, 0)),
                   pl.BlockSpec((BE, DD), lambda i: (i, 0)),
                   pl.BlockSpec((HH, BE), lambda i: (0, i))],
        out_shape=[jax.ShapeDtypeStruct((EE, DD), jnp.float32),
                   jax.ShapeDtypeStruct((EE, DD), jnp.float32),
                   jax.ShapeDtypeStruct((HH, EE), jnp.float32)],
    )(edge_attr, xd, qs, Wk, Wv, We, be2, shead)


def _sc_softmax(att_flat, src):
    mesh = plsc.VectorSubcoreMesh(core_axis_name="c", subcore_axis_name="s",
                                  num_cores=1)
    ET = EE // NS
    CH = 128
    NF = ET // CH
    TL = ET - NF * CH
    ZT = (HH * NN) // NS // 8 * 8
    ZR = HH * NN - ZT * NS

    @functools.partial(
        pl.kernel,
        out_type=jax.ShapeDtypeStruct((HH * EE,), jnp.float32),
        mesh=mesh,
        scratch_types=[
            pltpu.VMEM((HH * ET,), jnp.float32),
            pltpu.VMEM((HH * NN,), jnp.float32),
            pltpu.VMEM((ET,), jnp.int32),
            pltpu.VMEM((CH,), jnp.int32),
            pltpu.VMEM((TL,), jnp.int32),
            pltpu.VMEM_SHARED((HH * NN,), jnp.float32),
            pltpu.VMEM_SHARED((HH * NN,), jnp.float32),
        ],
        compiler_params=pltpu.CompilerParams(needs_layout_passes=False),
    )
    def k(att_hbm, src_hbm, coeff_hbm, av, tab, srcv, idx, idxt,
          mtab_s, den_s):
        t = lax.axis_index("s")
        ebase = t * ET

        zv = jnp.zeros((LANES,), jnp.float32)

        def zbody(g, carry):
            av[pl.ds(g * LANES, LANES)] = zv
            return carry
        lax.fori_loop(0, ZT // LANES, zbody, 0)
        pltpu.sync_copy(av.at[pl.ds(0, ZT)], den_s.at[pl.ds(t * ZT, ZT)])

        @pl.when(t == NS - 1)
        def _():
            pltpu.sync_copy(av.at[pl.ds(0, ZR)],
                            den_s.at[pl.ds(NS * ZT, ZR)])

        pltpu.sync_copy(src_hbm.at[pl.ds(ebase, ET)], srcv)
        for h in range(HH):
            pltpu.sync_copy(att_hbm.at[pl.ds(h * EE + ebase, ET)],
                            av.at[pl.ds(h * ET, ET)])

        def build_idx(off, h, idxr, n):
            for j in range(n // LANES):
                s16 = srcv[pl.ds(off + j * LANES, LANES)]
                idxr[pl.ds(j * LANES, LANES)] = s16 + h * NN

        for h in range(HH):
            def mscat(i, carry, h=h):
                off = i * CH
                build_idx(off, h, idx, CH)
                pltpu.sync_copy(av.at[pl.ds(h * ET + off, CH)],
                                mtab_s.at[idx])
                return carry
            lax.fori_loop(0, NF, mscat, 0)
            build_idx(NF * CH, h, idxt, TL)
            pltpu.sync_copy(av.at[pl.ds(h * ET + NF * CH, TL)],
                            mtab_s.at[idxt])

        plsc.subcore_barrier()
        pltpu.sync_copy(mtab_s, tab)

        for h in range(HH):
            def exbody(g, carry, h=h):
                s16 = srcv[pl.ds(g * LANES, LANES)]
                m = plsc.load_gather(tab, [s16 + h * NN])
                a = av[pl.ds(h * ET + g * LANES, LANES)]
                av[pl.ds(h * ET + g * LANES, LANES)] = jnp.exp(a - m)
                return carry
            lax.fori_loop(0, ET // LANES, exbody, 0)

        for h in range(HH):
            def dscat(i, carry, h=h):
                off = i * CH
                build_idx(off, h, idx, CH)
                pltpu.sync_copy(av.at[pl.ds(h * ET + off, CH)],
                                den_s.at[idx], add=True)
                return carry
            lax.fori_loop(0, NF, dscat, 0)
            build_idx(NF * CH, h, idxt, TL)
            pltpu.sync_copy(av.at[pl.ds(h * ET + NF * CH, TL)],
                            den_s.at[idxt], add=True)

        plsc.subcore_barrier()
        pltpu.sync_copy(den_s, tab)

        for h in range(HH):
            def cbody(g, carry, h=h):
                s16 = srcv[pl.ds(g * LANES, LANES)]
                d = plsc.load_gather(tab, [s16 + h * NN])
                e = av[pl.ds(h * ET + g * LANES, LANES)]
                av[pl.ds(h * ET + g * LANES, LANES)] = e / (d + 1e-16)
                return carry
            lax.fori_loop(0, ET // LANES, cbody, 0)
        for h in range(HH):
            pltpu.sync_copy(av.at[pl.ds(h * ET, ET)],
                            coeff_hbm.at[pl.ds(h * EE + ebase, ET)])

    return k(att_flat, src)


def _w_body(coeff_ref, v_ref, ex_ref, w_ref):
    scale = jnp.dot(coeff_ref[...].T, ex_ref[...],
                    preferred_element_type=jnp.float32)
    w = v_ref[...] * scale
    w_ref[:, 0, :] = w[:, :DD // 2]
    w_ref[:, 1, :] = w[:, DD // 2:]


def _tc_scale(coeff_t, v):
    BE = 640
    expand = jnp.repeat(jnp.eye(HH, dtype=jnp.float32), CC, axis=1)
    return pl.pallas_call(
        _w_body,
        grid=(EE // BE,),
        in_specs=[pl.BlockSpec((HH, BE), lambda i: (0, i)),
                  pl.BlockSpec((BE, DD), lambda i: (i, 0)),
                  pl.BlockSpec((HH, DD), lambda i: (0, 0))],
        out_specs=pl.BlockSpec((BE, 2, DD // 2), lambda i: (i, 0, 0)),
        out_shape=jax.ShapeDtypeStruct((EE, 2, DD // 2), jnp.float32),
    )(coeff_t, v, expand)


def _sc_scatter_out(w2, src, zeros2d):
    mesh = plsc.VectorSubcoreMesh(core_axis_name="c", subcore_axis_name="s")
    ET = EE // NS
    CH = 128
    NF = ET // CH
    TL = ET - NF * CH
    RT = 624
    HW = DD // 2

    @functools.partial(
        pl.kernel,
        out_type=jax.ShapeDtypeStruct((NN, 2, HW), jnp.float32),
        mesh=mesh,
        scratch_types=[
            pltpu.VMEM((CH, HW), jnp.float32),
            pltpu.VMEM((TL, HW), jnp.float32),
            pltpu.VMEM((CH,), jnp.int32),
            pltpu.VMEM((TL,), jnp.int32),
            pltpu.VMEM_SHARED((NN, HW), jnp.float32),
        ],
    )
    def k(w_hbm, src_hbm, z_hbm, out_hbm, wbuf, wbuft, idx, idxt, acc_s):
        c = lax.axis_index("c")
        t = lax.axis_index("s")

        pltpu.sync_copy(z_hbm.at[pl.ds(0, RT), :],
                        acc_s.at[pl.ds(t * RT, RT), :])

        @pl.when(t == NS - 1)
        def _():
            pltpu.sync_copy(z_hbm.at[pl.ds(0, 16), :],
                            acc_s.at[pl.ds(NS * RT, 16), :])

        plsc.subcore_barrier()

        base = t * ET

        def scat(i, carry):
            off = base + i * CH
            pltpu.sync_copy(src_hbm.at[pl.ds(off, CH)], idx)
            pltpu.sync_copy(w_hbm.at[pl.ds(off, CH), c, :], wbuf)
            pltpu.sync_copy(wbuf, acc_s.at[idx], add=True)
            return carry
        lax.fori_loop(0, NF, scat, 0)
        offt = base + NF * CH
        pltpu.sync_copy(src_hbm.at[pl.ds(offt, TL)], idxt)
        pltpu.sync_copy(w_hbm.at[pl.ds(offt, TL), c, :], wbuft)
        pltpu.sync_copy(wbuft, acc_s.at[idxt], add=True)

        plsc.subcore_barrier()

        pltpu.sync_copy(acc_s.at[pl.ds(t * RT, RT), :],
                        out_hbm.at[pl.ds(t * RT, RT), c, :])

        @pl.when(t == NS - 1)
        def _():
            pltpu.sync_copy(acc_s.at[pl.ds(NS * RT, 16), :],
                            out_hbm.at[pl.ds(NS * RT, 16), c, :])

    return k(w2, src, zeros2d)


def kernel(x, edge_index, edge_attr, Wq, Wk, Wv, We, be):
    src = edge_index[0]
    dst = edge_index[1]
    query = _tc_query(x, Wq)
    xd, qs = _sc_gather(x, query, dst, src)
    v, eout, att_t = _tc_proj(edge_attr, xd, qs, Wk, Wv, We, be)
    coeff_flat = _sc_softmax(att_t.reshape(HH * EE), src)
    w2 = _tc_scale(coeff_flat.reshape(HH, EE), v)
    z2 = jnp.zeros((624, DD // 2), jnp.float32)
    out2 = _sc_scatter_out(w2, src, z2)
    return out2.reshape(NN, DD), eout


# speedup vs baseline: 11.3147x; 1.1239x over previous
import functools
import math

import jax
import jax.numpy as jnp
from jax import lax
from jax.experimental import pallas as pl
from jax.experimental.pallas import tpu as pltpu
from jax.experimental.pallas import tpu_sc as plsc

NN = 10000
EE = 160000
DD = 256
HH = 4
CC = DD // HH

NC = 2
NS = 16
LANES = 16


def _q_body(x_ref, wq_ref, q_ref):
    q_ref[...] = jnp.dot(x_ref[...], wq_ref[...],
                         preferred_element_type=jnp.float32)


def _tc_query(x, Wq):
    BN = 1000
    return pl.pallas_call(
        _q_body,
        grid=(NN // BN,),
        in_specs=[pl.BlockSpec((BN, DD), lambda i: (i, 0)),
                  pl.BlockSpec((DD, DD), lambda i: (0, 0))],
        out_specs=pl.BlockSpec((BN, DD), lambda i: (i, 0)),
        out_shape=jax.ShapeDtypeStruct((NN, DD), jnp.float32),
    )(x, Wq)


def _sc_gather(x, query, dst, src):
    mesh = plsc.VectorSubcoreMesh(core_axis_name="c", subcore_axis_name="s")
    EW = EE // (NC * NS)
    CH = 128
    NF = EW // CH
    TL = EW - NF * CH

    NB = 3
    NFI = NF // NB

    @functools.partial(
        pl.kernel,
        out_type=(jax.ShapeDtypeStruct((EE, DD), jnp.float32),
                  jax.ShapeDtypeStruct((EE, DD), jnp.float32)),
        mesh=mesh,
        scratch_types=[pltpu.VMEM((EW,), jnp.int32),
                       pltpu.VMEM((EW,), jnp.int32),
                       pltpu.VMEM((CH, DD), jnp.float32),
                       pltpu.VMEM((CH, DD), jnp.float32),
                       pltpu.VMEM((CH, DD), jnp.float32),
                       pltpu.SemaphoreType.DMA,
                       pltpu.SemaphoreType.DMA,
                       pltpu.SemaphoreType.DMA,
                       pltpu.SemaphoreType.DMA,
                       pltpu.SemaphoreType.DMA,
                       pltpu.SemaphoreType.DMA],
    )
    def k(x_hbm, q_hbm, dst_hbm, src_hbm, xd_hbm, qs_hbm,
          idxd, idxs, b0, b1, b2, g0, g1, g2, w0, w1, w2):
        wid = lax.axis_index("s") * NC + lax.axis_index("c")
        base = wid * EW
        bufs = (b0, b1, b2)
        gsem = (g0, g1, g2)
        wsem = (w0, w1, w2)

        pltpu.sync_copy(dst_hbm.at[pl.ds(base, EW)], idxd)
        pltpu.sync_copy(src_hbm.at[pl.ds(base, EW)], idxs)

        def run(tab_hbm, idxall, out_hbm):
            def gather(c, j):
                pltpu.async_copy(tab_hbm.at[idxall.at[pl.ds(c * CH, CH)]],
                                 bufs[j], gsem[j])

            for j in range(NB):
                gather(j, j)

            def body(i, carry):
                for j in range(NB):
                    c = i * NB + j
                    pltpu.make_async_copy(
                        tab_hbm.at[idxall.at[pl.ds(c * CH, CH)]],
                        bufs[j], gsem[j]).wait()
                    pltpu.async_copy(bufs[j],
                                     out_hbm.at[pl.ds(base + c * CH, CH)],
                                     wsem[j])

                    @pl.when(i < NFI - 1)
                    def _(j=j, c=c):
                        pltpu.make_async_copy(
                            bufs[j],
                            out_hbm.at[pl.ds(base + c * CH, CH)],
                            wsem[j]).wait()
                        gather(c + NB, j)
                return carry

            lax.fori_loop(0, NFI, body, 0)
            for j in range(NB):
                pltpu.make_async_copy(
                    bufs[j],
                    out_hbm.at[pl.ds(base + (NF - NB + j) * CH, CH)],
                    wsem[j]).wait()
            pltpu.async_copy(tab_hbm.at[idxall.at[pl.ds(NF * CH, TL)]],
                             bufs[0].at[pl.ds(0, TL), :], gsem[0])
            pltpu.make_async_copy(tab_hbm.at[idxall.at[pl.ds(NF * CH, TL)]],
                                  bufs[0].at[pl.ds(0, TL), :], gsem[0]).wait()
            pltpu.sync_copy(bufs[0].at[pl.ds(0, TL), :],
                            out_hbm.at[pl.ds(base + NF * CH, TL)])

        run(x_hbm, idxd, xd_hbm)
        run(q_hbm, idxs, qs_hbm)

    return k(x, query, dst, src)


def _att_body(ea_ref, xd_ref, qs_ref, wk_ref, wv_ref, we_ref, be_ref,
              sh_ref, v_ref, eo_ref, att_ref):
    z = (ea_ref[...] * xd_ref[...]).astype(jnp.bfloat16)
    kk = jnp.dot(z, wk_ref[...], preferred_element_type=jnp.float32)
    v_ref[...] = jnp.dot(z, wv_ref[...], preferred_element_type=jnp.float32)
    eo_ref[...] = (jnp.dot(z, we_ref[...], preferred_element_type=jnp.float32)
                   + be_ref[...])
    p = (qs_ref[...] * kk).astype(jnp.bfloat16)
    att = jnp.dot(p, sh_ref[...],
                  preferred_element_type=jnp.float32)
    att_ref[...] = att.T


def _tc_proj(edge_attr, xd, qs, Wk, Wv, We, be):
    BE = 640
    Wk = Wk.astype(jnp.bfloat16)
    Wv = Wv.astype(jnp.bfloat16)
    We = We.astype(jnp.bfloat16)
    shead = (jnp.repeat(jnp.eye(HH, dtype=jnp.float32), CC, axis=0)
             * (1.0 / math.sqrt(CC))).astype(jnp.bfloat16)
    be2 = be.reshape(1, DD)
    return pl.pallas_call(
        _att_body,
        grid=(EE // BE,),
        in_specs=[pl.BlockSpec((BE, DD), lambda i: (i, 0)),
                  pl.BlockSpec((BE, DD), lambda i: (i, 0)),
                  pl.BlockSpec((BE, DD), lambda i: (i, 0)),
                  pl.BlockSpec((DD, DD), lambda i: (0, 0)),
                  pl.BlockSpec((DD, DD), lambda i: (0, 0)),
                  pl.BlockSpec((DD, DD), lambda i: (0, 0)),
                  pl.BlockSpec((1, DD), lambda i: (0, 0)),
                  pl.BlockSpec((DD, HH), lambda i: (0, 0))],
        out_specs=[pl.BlockSpec((BE, DD), lambda i: (i, 0)),
                   pl.BlockSpec((BE, DD), lambda i: (i, 0)),
                   pl.BlockSpec((HH, BE), lambda i: (0, i))],
        out_shape=[jax.ShapeDtypeStruct((EE, DD), jnp.float32),
                   jax.ShapeDtypeStruct((EE, DD), jnp.float32),
                   jax.ShapeDtypeStruct((HH, EE), jnp.float32)],
    )(edge_attr, xd, qs, Wk, Wv, We, be2, shead)


def _sc_softmax(att_flat, src):
    mesh = plsc.VectorSubcoreMesh(core_axis_name="c", subcore_axis_name="s",
                                  num_cores=1)
    ET = EE // NS
    CH = 128
    NF = ET // CH
    TL = ET - NF * CH
    ZT = (HH * NN) // NS // 8 * 8
    ZR = HH * NN - ZT * NS

    @functools.partial(
        pl.kernel,
        out_type=jax.ShapeDtypeStruct((HH * EE,), jnp.float32),
        mesh=mesh,
        scratch_types=[
            pltpu.VMEM((HH * ET,), jnp.float32),
            pltpu.VMEM((HH * NN,), jnp.float32),
            pltpu.VMEM((ET,), jnp.int32),
            pltpu.VMEM((CH,), jnp.int32),
            pltpu.VMEM((TL,), jnp.int32),
            pltpu.VMEM_SHARED((HH * NN,), jnp.float32),
            pltpu.VMEM_SHARED((HH * NN,), jnp.float32),
        ],
        compiler_params=pltpu.CompilerParams(needs_layout_passes=False),
    )
    def k(att_hbm, src_hbm, coeff_hbm, av, tab, srcv, idx, idxt,
          mtab_s, den_s):
        t = lax.axis_index("s")
        ebase = t * ET

        zv = jnp.zeros((LANES,), jnp.float32)

        def zbody(g, carry):
            av[pl.ds(g * LANES, LANES)] = zv
            return carry
        lax.fori_loop(0, ZT // LANES, zbody, 0)
        pltpu.sync_copy(av.at[pl.ds(0, ZT)], den_s.at[pl.ds(t * ZT, ZT)])

        @pl.when(t == NS - 1)
        def _():
            pltpu.sync_copy(av.at[pl.ds(0, ZR)],
                            den_s.at[pl.ds(NS * ZT, ZR)])

        pltpu.sync_copy(src_hbm.at[pl.ds(ebase, ET)], srcv)
        for h in range(HH):
            pltpu.sync_copy(att_hbm.at[pl.ds(h * EE + ebase, ET)],
                            av.at[pl.ds(h * ET, ET)])

        def build_idx(off, h, idxr, n):
            for j in range(n // LANES):
                s16 = srcv[pl.ds(off + j * LANES, LANES)]
                idxr[pl.ds(j * LANES, LANES)] = s16 + h * NN

        for h in range(HH):
            def mscat(i, carry, h=h):
                off = i * CH
                build_idx(off, h, idx, CH)
                pltpu.sync_copy(av.at[pl.ds(h * ET + off, CH)],
                                mtab_s.at[idx])
                return carry
            lax.fori_loop(0, NF, mscat, 0)
            build_idx(NF * CH, h, idxt, TL)
            pltpu.sync_copy(av.at[pl.ds(h * ET + NF * CH, TL)],
                            mtab_s.at[idxt])

        plsc.subcore_barrier()
        pltpu.sync_copy(mtab_s, tab)

        for h in range(HH):
            def exbody(g, carry, h=h):
                s16 = srcv[pl.ds(g * LANES, LANES)]
                m = plsc.load_gather(tab, [s16 + h * NN])
                a = av[pl.ds(h * ET + g * LANES, LANES)]
                av[pl.ds(h * ET + g * LANES, LANES)] = jnp.exp(a - m)
                return carry
            lax.fori_loop(0, ET // LANES, exbody, 0)

        for h in range(HH):
            def dscat(i, carry, h=h):
                off = i * CH
                build_idx(off, h, idx, CH)
                pltpu.sync_copy(av.at[pl.ds(h * ET + off, CH)],
                                den_s.at[idx], add=True)
                return carry
            lax.fori_loop(0, NF, dscat, 0)
            build_idx(NF * CH, h, idxt, TL)
            pltpu.sync_copy(av.at[pl.ds(h * ET + NF * CH, TL)],
                            den_s.at[idxt], add=True)

        plsc.subcore_barrier()
        pltpu.sync_copy(den_s, tab)

        for h in range(HH):
            def cbody(g, carry, h=h):
                s16 = srcv[pl.ds(g * LANES, LANES)]
                d = plsc.load_gather(tab, [s16 + h * NN])
                e = av[pl.ds(h * ET + g * LANES, LANES)]
                av[pl.ds(h * ET + g * LANES, LANES)] = e / (d + 1e-16)
                return carry
            lax.fori_loop(0, ET // LANES, cbody, 0)
        for h in range(HH):
            pltpu.sync_copy(av.at[pl.ds(h * ET, ET)],
                            coeff_hbm.at[pl.ds(h * EE + ebase, ET)])

    return k(att_flat, src)


def _w_body(coeff_ref, v_ref, ex_ref, w_ref):
    scale = jnp.dot(coeff_ref[...], ex_ref[...],
                    preferred_element_type=jnp.float32)
    w = v_ref[...] * scale
    w_ref[:, 0, :] = w[:, :DD // 2]
    w_ref[:, 1, :] = w[:, DD // 2:]


def _tc_scale(coeff, v):
    BE = 2000
    expand = jnp.repeat(jnp.eye(HH, dtype=jnp.float32), CC, axis=1)
    return pl.pallas_call(
        _w_body,
        grid=(EE // BE,),
        in_specs=[pl.BlockSpec((BE, HH), lambda i: (i, 0)),
                  pl.BlockSpec((BE, DD), lambda i: (i, 0)),
                  pl.BlockSpec((HH, DD), lambda i: (0, 0))],
        out_specs=pl.BlockSpec((BE, 2, DD // 2), lambda i: (i, 0, 0)),
        out_shape=jax.ShapeDtypeStruct((EE, 2, DD // 2), jnp.float32),
    )(coeff, v, expand)


def _sc_scatter_out(w2, src, zeros2d):
    mesh = plsc.VectorSubcoreMesh(core_axis_name="c", subcore_axis_name="s")
    ET = EE // NS
    CH = 128
    NF = ET // CH
    TL = ET - NF * CH
    RT = 624
    HW = DD // 2

    @functools.partial(
        pl.kernel,
        out_type=jax.ShapeDtypeStruct((NN, 2, HW), jnp.float32),
        mesh=mesh,
        scratch_types=[
            pltpu.VMEM((CH, HW), jnp.float32),
            pltpu.VMEM((TL, HW), jnp.float32),
            pltpu.VMEM((CH,), jnp.int32),
            pltpu.VMEM((TL,), jnp.int32),
            pltpu.VMEM_SHARED((NN, HW), jnp.float32),
        ],
    )
    def k(w_hbm, src_hbm, z_hbm, out_hbm, wbuf, wbuft, idx, idxt, acc_s):
        c = lax.axis_index("c")
        t = lax.axis_index("s")

        pltpu.sync_copy(z_hbm.at[pl.ds(0, RT), :],
                        acc_s.at[pl.ds(t * RT, RT), :])

        @pl.when(t == NS - 1)
        def _():
            pltpu.sync_copy(z_hbm.at[pl.ds(0, 16), :],
                            acc_s.at[pl.ds(NS * RT, 16), :])

        plsc.subcore_barrier()

        base = t * ET

        def scat(i, carry):
            off = base + i * CH
            pltpu.sync_copy(src_hbm.at[pl.ds(off, CH)], idx)
            pltpu.sync_copy(w_hbm.at[pl.ds(off, CH), c, :], wbuf)
            pltpu.sync_copy(wbuf, acc_s.at[idx], add=True)
            return carry
        lax.fori_loop(0, NF, scat, 0)
        offt = base + NF * CH
        pltpu.sync_copy(src_hbm.at[pl.ds(offt, TL)], idxt)
        pltpu.sync_copy(w_hbm.at[pl.ds(offt, TL), c, :], wbuft)
        pltpu.sync_copy(wbuft, acc_s.at[idxt], add=True)

        plsc.subcore_barrier()

        pltpu.sync_copy(acc_s.at[pl.ds(t * RT, RT), :],
                        out_hbm.at[pl.ds(t * RT, RT), c, :])

        @pl.when(t == NS - 1)
        def _():
            pltpu.sync_copy(acc_s.at[pl.ds(NS * RT, 16), :],
                            out_hbm.at[pl.ds(NS * RT, 16), c, :])

    return k(w2, src, zeros2d)


def kernel(x, edge_index, edge_attr, Wq, Wk, Wv, We, be):
    src = edge_index[0]
    dst = edge_index[1]
    query = _tc_query(x, Wq)
    xd, qs = _sc_gather(x, query, dst, src)
    v, eout, att_t = _tc_proj(edge_attr, xd, qs, Wk, Wv, We, be)
    coeff_flat = _sc_softmax(att_t.reshape(HH * EE), src)
    w2 = _tc_scale(coeff_flat.reshape(HH, EE).T, v)
    z2 = jnp.zeros((624, DD // 2), jnp.float32)
    out2 = _sc_scatter_out(w2, src, z2)
    return out2.reshape(NN, DD), eout
